# baseline (speedup 1.0000x reference)
import functools

import jax
import jax.numpy as jnp
from jax import lax
from jax.experimental import pallas as pl
from jax.experimental.pallas import tpu as pltpu
from jax.experimental.pallas import tpu_sc as plsc

_D = 128
_E = 320000
_N = 10000
_C = 40
_NW = 32
_EPW = _E // _NW
_R = 5
_CPT = _R * _C
_T = _EPW // _CPT
_KH = 3
_TROWS = 640

_mesh = plsc.VectorSubcoreMesh(core_axis_name="c", subcore_axis_name="s")


@functools.partial(
    pl.kernel,
    out_type=jax.ShapeDtypeStruct((_E, _D), jnp.float32),
    mesh=_mesh,
    scratch_types=[
        pltpu.VMEM((2, _R, _C), jnp.int32),
        pltpu.VMEM((_R, _C, _D), jnp.float32),
        pltpu.VMEM_SHARED((_N, _D), jnp.float32),
        pltpu.SemaphoreType.DMA((2,)),
        pltpu.SemaphoreType.DMA((_R,)),
        pltpu.SemaphoreType.DMA((_R,)),
        pltpu.SemaphoreType.DMA,
    ],
)
def _gather_kernel(table, idx_hbm, out, idxr, rbuf, shtab, isem, gsem, wsem,
                   ssem):
    sid = lax.axis_index("s")
    wid = sid * 2 + lax.axis_index("c")
    e0 = wid * _EPW

    def start_idx(trip, par):
        pltpu.async_copy(
            idx_hbm.at[wid, trip], idxr.at[par], isem.at[par]
        )

    def wait_idx(par):
        pltpu.make_async_copy(
            idx_hbm.at[0, 0], idxr.at[0], isem.at[par]
        ).wait()

    def start_gather(par, b, src):
        pltpu.async_copy(
            src.at[idxr.at[par, b]], rbuf.at[b], gsem.at[b]
        )

    def wait_gather(b):
        pltpu.make_async_copy(
            shtab.at[idxr.at[0, 0]], rbuf.at[b], gsem.at[b]
        ).wait()

    def start_write(c, b):
        pltpu.async_copy(
            rbuf.at[b], out.at[pl.ds(e0 + c * _C, _C)], wsem.at[b]
        )

    def wait_write(b):
        pltpu.make_async_copy(
            rbuf.at[b], out.at[pl.ds(0, _C)], wsem.at[b]
        ).wait()

    start_idx(0, 0)
    off = jnp.where(sid == 15, _N - _TROWS, sid * _TROWS)
    off = pl.multiple_of(off, 16)
    pltpu.async_copy(
        table.at[pl.ds(off, _TROWS)], shtab.at[pl.ds(off, _TROWS)], ssem
    )

    def make_trip(src):
        def trip(t, carry):
            par = lax.rem(t, 2)
            c0 = t * _R

            wait_idx(par)

            @pl.when(t < _T - 1)
            def _():
                start_idx(t + 1, 1 - par)

            for b in range(_R):
                @pl.when(t > 0)
                def _():
                    wait_write(b)

                start_gather(par, b, src)

            for b in range(_R):
                wait_gather(b)
                start_write(c0 + b, b)
            return carry
        return trip

    lax.fori_loop(0, _KH, make_trip(table), 0)
    pltpu.make_async_copy(
        table.at[pl.ds(off, _TROWS)], shtab.at[pl.ds(off, _TROWS)], ssem
    ).wait()
    plsc.subcore_barrier()
    lax.fori_loop(_KH, _T, make_trip(shtab), 0)

    for b in range(_R):
        wait_write(b)


def kernel(src_node_states, dst_node_states, dst_index, src_index, edge_states):
    del dst_node_states, dst_index, edge_states
    idx4 = src_index.reshape(_NW, _T, _R, _C)
    return _gather_kernel(src_node_states, idx4)


# speedup vs baseline: 1.0066x; 1.0066x over previous
import functools

import jax
import jax.numpy as jnp
from jax import lax
from jax.experimental import pallas as pl
from jax.experimental.pallas import tpu as pltpu
from jax.experimental.pallas import tpu_sc as plsc

_D = 128
_E = 320000
_N = 10000
_C = 40
_NW = 32
_EPW = _E // _NW
_CPW = _EPW // _C
_R = 5
_LAG = 4
_RL = 4
_ILAG = 1
_T = _CPW // _R
_KH = 3
_TROWS = 640

_mesh = plsc.VectorSubcoreMesh(core_axis_name="c", subcore_axis_name="s")


@functools.partial(
    pl.kernel,
    out_type=jax.ShapeDtypeStruct((_E, _D), jnp.float32),
    mesh=_mesh,
    scratch_types=[
        pltpu.VMEM((_R, _C), jnp.int32),
        pltpu.VMEM((_R, _C, _D), jnp.float32),
        pltpu.VMEM_SHARED((_N, _D), jnp.float32),
        pltpu.SemaphoreType.DMA((_R,)),
        pltpu.SemaphoreType.DMA((_R,)),
        pltpu.SemaphoreType.DMA((_R,)),
        pltpu.SemaphoreType.DMA,
    ],
)
def _gather_kernel(table, idx_hbm, out, idxr, rbuf, shtab, isem, gsem, wsem,
                   ssem):
    sid = lax.axis_index("s")
    wid = sid * 2 + lax.axis_index("c")
    e0 = wid * _EPW

    def start_idx(c, slot):
        pltpu.async_copy(
            idx_hbm.at[pl.ds(e0 + c * _C, _C)], idxr.at[slot], isem.at[slot]
        )

    def wait_idx(slot):
        pltpu.make_async_copy(
            idx_hbm.at[pl.ds(0, _C)], idxr.at[slot], isem.at[slot]
        ).wait()

    def start_gather(slot, src):
        pltpu.async_copy(src.at[idxr.at[slot]], rbuf.at[slot], gsem.at[slot])

    def wait_gather(slot):
        pltpu.make_async_copy(
            shtab.at[idxr.at[0]], rbuf.at[slot], gsem.at[slot]
        ).wait()

    def start_write(c, slot):
        pltpu.async_copy(
            rbuf.at[slot], out.at[pl.ds(e0 + c * _C, _C)], wsem.at[slot]
        )

    def wait_write(slot):
        pltpu.make_async_copy(
            rbuf.at[slot], out.at[pl.ds(0, _C)], wsem.at[slot]
        ).wait()

    for s in range(_RL):
        start_idx(s, s)

    off = jnp.where(sid == 15, _N - _TROWS, sid * _TROWS)
    off = pl.multiple_of(off, 16)
    pltpu.async_copy(
        table.at[pl.ds(off, _TROWS)], shtab.at[pl.ds(off, _TROWS)], ssem
    )

    for s in range(_ILAG):
        wait_idx(s)
        start_gather(s, table)

    def make_outer(src):
        def outer(t, carry):
            for b in range(_R):
                c = t * _R + b
                slot = b
                slot_w = (b + _R - _LAG) % _R
                slot_i = (b + _RL) % _R
                slot_g = (b + _ILAG) % _R

                if b >= _LAG:
                    wait_write(slot_w)
                else:
                    @pl.when(t > 0)
                    def _():
                        wait_write(slot_w)

                if b + _RL < _R:
                    start_idx(c + _RL, slot_i)
                else:
                    @pl.when(t < _T - 1)
                    def _():
                        start_idx(c + _RL, slot_i)

                if b >= _R - _ILAG:
                    @pl.when(t < _T - 1)
                    def _():
                        wait_idx(slot_g)
                        start_gather(slot_g, src)
                else:
                    wait_idx(slot_g)
                    start_gather(slot_g, src)

                wait_gather(slot)
                start_write(c, slot)
            return carry
        return outer

    lax.fori_loop(0, _KH, make_outer(table), 0)
    pltpu.make_async_copy(
        table.at[pl.ds(off, _TROWS)], shtab.at[pl.ds(off, _TROWS)], ssem
    ).wait()
    plsc.subcore_barrier()
    lax.fori_loop(_KH, _T, make_outer(shtab), 0)

    for s in range(_R - _LAG, _R):
        wait_write(s)


def kernel(src_node_states, dst_node_states, dst_index, src_index, edge_states):
    del dst_node_states, dst_index, edge_states
    return _gather_kernel(src_node_states, src_index)


# speedup vs baseline: 1.0536x; 1.0467x over previous
import functools

import jax
import jax.numpy as jnp
from jax import lax
from jax.experimental import pallas as pl
from jax.experimental.pallas import tpu as pltpu
from jax.experimental.pallas import tpu_sc as plsc

_D = 128
_E = 320000
_N = 10000
_C = 40
_NW = 32
_EPW = _E // _NW
_CPW = _EPW // _C
_R = 5
_LAG = 3
_RL = 4
_ILAG = 2
_T = _CPW // _R
_KH = 2
_TROWS = 640

_mesh = plsc.VectorSubcoreMesh(core_axis_name="c", subcore_axis_name="s")


@functools.partial(
    pl.kernel,
    out_type=jax.ShapeDtypeStruct((_E, _D), jnp.float32),
    mesh=_mesh,
    scratch_types=[
        pltpu.VMEM((_R, _C), jnp.int32),
        pltpu.VMEM((_R, _C, _D), jnp.float32),
        pltpu.VMEM_SHARED((_N, _D), jnp.float32),
        pltpu.SemaphoreType.DMA((_R,)),
        pltpu.SemaphoreType.DMA((_R,)),
        pltpu.SemaphoreType.DMA((_R,)),
        pltpu.SemaphoreType.DMA,
    ],
)
def _gather_kernel(table, idx_hbm, out, idxr, rbuf, shtab, isem, gsem, wsem,
                   ssem):
    sid = lax.axis_index("s")
    wid = sid * 2 + lax.axis_index("c")
    e0 = wid * _EPW

    def start_idx(c, slot):
        pltpu.async_copy(
            idx_hbm.at[pl.ds(e0 + c * _C, _C)], idxr.at[slot], isem.at[slot]
        )

    def wait_idx(slot):
        pltpu.make_async_copy(
            idx_hbm.at[pl.ds(0, _C)], idxr.at[slot], isem.at[slot]
        ).wait()

    def start_gather(slot, src):
        pltpu.async_copy(src.at[idxr.at[slot]], rbuf.at[slot], gsem.at[slot])

    def wait_gather(slot):
        pltpu.make_async_copy(
            shtab.at[idxr.at[0]], rbuf.at[slot], gsem.at[slot]
        ).wait()

    def start_write(c, slot):
        pltpu.async_copy(
            rbuf.at[slot], out.at[pl.ds(e0 + c * _C, _C)], wsem.at[slot]
        )

    def wait_write(slot):
        pltpu.make_async_copy(
            rbuf.at[slot], out.at[pl.ds(0, _C)], wsem.at[slot]
        ).wait()

    for s in range(_RL):
        start_idx(s, s)

    off = jnp.where(sid == 15, _N - _TROWS, sid * _TROWS)
    off = pl.multiple_of(off, 16)
    pltpu.async_copy(
        table.at[pl.ds(off, _TROWS)], shtab.at[pl.ds(off, _TROWS)], ssem
    )

    for s in range(_ILAG):
        wait_idx(s)
        start_gather(s, table)

    def make_outer(src):
        def outer(t, carry):
            for b in range(_R):
                c = t * _R + b
                slot = b
                slot_w = (b + _R - _LAG) % _R
                slot_i = (b + _RL) % _R
                slot_g = (b + _ILAG) % _R

                if b >= _LAG:
                    wait_write(slot_w)
                else:
                    @pl.when(t > 0)
                    def _():
                        wait_write(slot_w)

                if b + _RL < _R:
                    start_idx(c + _RL, slot_i)
                else:
                    @pl.when(t < _T - 1)
                    def _():
                        start_idx(c + _RL, slot_i)

                if b >= _R - _ILAG:
                    @pl.when(t < _T - 1)
                    def _():
                        wait_idx(slot_g)
                        start_gather(slot_g, src)
                else:
                    wait_idx(slot_g)
                    start_gather(slot_g, src)

                wait_gather(slot)
                start_write(c, slot)
            return carry
        return outer

    lax.fori_loop(0, _KH, make_outer(table), 0)
    pltpu.make_async_copy(
        table.at[pl.ds(off, _TROWS)], shtab.at[pl.ds(off, _TROWS)], ssem
    ).wait()
    plsc.subcore_barrier()
    lax.fori_loop(_KH, _T, make_outer(shtab), 0)

    for s in range(_R - _LAG, _R):
        wait_write(s)


def kernel(src_node_states, dst_node_states, dst_index, src_index, edge_states):
    del dst_node_states, dst_index, edge_states
    return _gather_kernel(src_node_states, src_index)
